# baseline (speedup 1.0000x reference)
import functools

import jax
import jax.numpy as jnp
from jax import lax
from jax.experimental import pallas as pl
from jax.experimental.pallas import tpu as pltpu
from jax.experimental.pallas import tpu_sc as plsc

DIM = 128
BATCH = 16384
NW = 32
CHUNK = 128
ROWS_PER_W = BATCH // NW
NCHUNK = ROWS_PER_W // CHUNK
NBLK = CHUNK // 16
NSUB = DIM // 16


def _treesum(xs):
    xs = list(xs)
    while len(xs) > 1:
        nxt = [a + b for a, b in zip(xs[::2], xs[1::2])]
        if len(xs) % 2:
            nxt.append(xs[-1])
        xs = nxt
    return xs[0]


def _rsqrt(x):
    i = lax.bitcast_convert_type(x, jnp.int32)
    i = jnp.int32(0x5F3759DF) - lax.shift_right_arithmetic(i, 1)
    y = lax.bitcast_convert_type(i, jnp.float32)
    xh = jnp.float32(0.5) * x
    for _ in range(2):
        y = y * (jnp.float32(1.5) - xh * y * y)
    return y


def _sumsq(vs):
    acc = vs[0] * vs[0]
    for v in vs[1:]:
        acc = acc + v * v
    return jnp.maximum(jnp.sum(acc), jnp.float32(1e-24))


def _make_kernel():
    mesh = plsc.VectorSubcoreMesh(core_axis_name="c", subcore_axis_name="s")

    @functools.partial(
        pl.kernel,
        mesh=mesh,
        compiler_params=pltpu.CompilerParams(needs_layout_passes=False),
        out_type=jax.ShapeDtypeStruct((NW, NCHUNK, CHUNK), jnp.float32),
        scratch_types=[
            pltpu.VMEM((NCHUNK, CHUNK), jnp.int32),
            pltpu.VMEM((NCHUNK, CHUNK), jnp.int32),
            pltpu.VMEM((NCHUNK, CHUNK), jnp.int32),
            pltpu.VMEM((2, CHUNK, DIM), jnp.float32),
            pltpu.VMEM((2, CHUNK, DIM), jnp.float32),
            pltpu.VMEM((2, CHUNK, DIM), jnp.float32),
            pltpu.VMEM((CHUNK, 17), jnp.float32),
            pltpu.VMEM((NCHUNK, CHUNK), jnp.float32),
            pltpu.SemaphoreType.DMA,
            pltpu.SemaphoreType.DMA,
            pltpu.SemaphoreType.DMA,
            pltpu.SemaphoreType.DMA,
            pltpu.SemaphoreType.DMA,
            pltpu.SemaphoreType.DMA,
        ],
    )
    def trans_score(h_hbm, t_hbm, r_hbm, tail_hbm, rel_hbm, out_hbm,
                    hidx, tidx, ridx, hbuf, tbuf, rbuf, pb_s, score,
                    sh0, st0, sr0, sh1, st1, sr1):
        wid = lax.axis_index("s") * 2 + lax.axis_index("c")
        pltpu.sync_copy(h_hbm.at[wid], hidx)
        pltpu.sync_copy(t_hbm.at[wid], tidx)
        pltpu.sync_copy(r_hbm.at[wid], ridx)

        lane = lax.iota(jnp.int32, 16)
        zero = jnp.zeros((16,), jnp.float32)
        sems = ((sh0, st0, sr0), (sh1, st1, sr1))

        def fire(c, p):
            dh = pltpu.async_copy(tail_hbm.at[hidx.at[c]], hbuf.at[p],
                                  sems[p][0])
            dt = pltpu.async_copy(tail_hbm.at[tidx.at[c]], tbuf.at[p],
                                  sems[p][1])
            dr = pltpu.async_copy(rel_hbm.at[ridx.at[c]], rbuf.at[p],
                                  sems[p][2])
            return (dh, dt, dr)

        pend = fire(0, 0)
        for c in range(NCHUNK):
            p = c % 2
            for d in pend:
                d.wait()
            if c + 1 < NCHUNK:
                pend = fire(c + 1, 1 - p)
            hb, tb, rb = hbuf.at[p], tbuf.at[p], rbuf.at[p]

            @plsc.parallel_loop(0, CHUNK, unroll=1)
            def row_fn(i, hb=hb, tb=tb, rb=rb):
                hv = [hb[i, pl.ds(s * 16, 16)] for s in range(NSUB)]
                tv = [tb[i, pl.ds(s * 16, 16)] for s in range(NSUB)]
                rv = [rb[i, pl.ds(s * 16, 16)] for s in range(NSUB)]
                sh = _sumsq(hv)
                ih = _rsqrt(sh)
                it = _rsqrt(_sumsq(tv))
                ir = _rsqrt(_sumsq(rv))
                q = sh * ih
                a = jnp.full((16,), q * ir, jnp.float32)
                b = jnp.full((16,), q * it, jnp.float32)
                acc = _treesum([
                    jnp.abs(hv[s] + rv[s] * a - tv[s] * b)
                    for s in range(NSUB)
                ])
                pb_s[i, pl.ds(0, 16)] = acc * jnp.full((16,), ih,
                                                       jnp.float32)

            def grp(b, _, c=c):
                sc = zero
                for j in range(16):
                    jv = jnp.full((16,), j, jnp.int32)
                    sc = sc + plsc.load_gather(pb_s, [b * 16 + lane, jv])
                score[c, pl.ds(b * 16, 16)] = sc
                return 0

            lax.fori_loop(0, NBLK, grp, 0)

        pltpu.sync_copy(score, out_hbm.at[wid])

    return trans_score


_sc_score = _make_kernel()


def kernel(batch_h, batch_t, batch_r, tail_emb, rel_emb):
    h = batch_h.astype(jnp.int32).reshape(NW, NCHUNK, CHUNK)
    t = batch_t.astype(jnp.int32).reshape(NW, NCHUNK, CHUNK)
    r = batch_r.astype(jnp.int32).reshape(NW, NCHUNK, CHUNK)
    out = _sc_score(h, t, r, tail_emb, rel_emb)
    return out.reshape(-1)


# speedup vs baseline: 1.1694x; 1.1694x over previous
import functools

import jax
import jax.numpy as jnp
from jax import lax
from jax.experimental import pallas as pl
from jax.experimental.pallas import tpu as pltpu
from jax.experimental.pallas import tpu_sc as plsc

DIM = 128
BATCH = 16384
NW = 32
CHUNK = 128
ROWS_PER_W = BATCH // NW
NCHUNK = ROWS_PER_W // CHUNK
NBLK = CHUNK // 16
NSUB = DIM // 16


def _treesum(xs):
    xs = list(xs)
    while len(xs) > 1:
        nxt = [a + b for a, b in zip(xs[::2], xs[1::2])]
        if len(xs) % 2:
            nxt.append(xs[-1])
        xs = nxt
    return xs[0]


def _rsqrt(x):
    i = lax.bitcast_convert_type(x, jnp.int32)
    i = jnp.int32(0x5F3759DF) - lax.shift_right_arithmetic(i, 1)
    y = lax.bitcast_convert_type(i, jnp.float32)
    xh = jnp.float32(0.5) * x
    for _ in range(3):
        y = y * (jnp.float32(1.5) - xh * y * y)
    return y


def _sumsq(vs):
    acc = _treesum([v * v for v in vs])
    return jnp.maximum(jnp.sum(acc), jnp.float32(1e-24))


def _make_kernel():
    mesh = plsc.VectorSubcoreMesh(core_axis_name="c", subcore_axis_name="s")

    @functools.partial(
        pl.kernel,
        mesh=mesh,
        compiler_params=pltpu.CompilerParams(needs_layout_passes=False),
        out_type=jax.ShapeDtypeStruct((NW, NCHUNK, CHUNK), jnp.float32),
        scratch_types=[
            pltpu.VMEM((NCHUNK, CHUNK), jnp.int32),
            pltpu.VMEM((NCHUNK, CHUNK), jnp.int32),
            pltpu.VMEM((NCHUNK, CHUNK), jnp.int32),
            pltpu.VMEM((2, CHUNK, DIM), jnp.float32),
            pltpu.VMEM((2, CHUNK, DIM), jnp.float32),
            pltpu.VMEM((2, CHUNK, DIM), jnp.float32),
            pltpu.VMEM((CHUNK, 17), jnp.float32),
            pltpu.VMEM((NCHUNK, CHUNK), jnp.float32),
            pltpu.SemaphoreType.DMA,
            pltpu.SemaphoreType.DMA,
            pltpu.SemaphoreType.DMA,
            pltpu.SemaphoreType.DMA,
            pltpu.SemaphoreType.DMA,
            pltpu.SemaphoreType.DMA,
        ],
    )
    def trans_score(h_hbm, t_hbm, r_hbm, tail_hbm, rel_hbm, out_hbm,
                    hidx, tidx, ridx, hbuf, tbuf, rbuf, pb_s, score,
                    sh0, st0, sr0, sh1, st1, sr1):
        wid = lax.axis_index("s") * 2 + lax.axis_index("c")
        pltpu.sync_copy(h_hbm.at[wid], hidx)
        pltpu.sync_copy(t_hbm.at[wid], tidx)
        pltpu.sync_copy(r_hbm.at[wid], ridx)

        lane = lax.iota(jnp.int32, 16)
        zero = jnp.zeros((16,), jnp.float32)
        sems = ((sh0, st0, sr0), (sh1, st1, sr1))

        def fire(c, p):
            dh = pltpu.async_copy(tail_hbm.at[hidx.at[c]], hbuf.at[p],
                                  sems[p][0])
            dt = pltpu.async_copy(tail_hbm.at[tidx.at[c]], tbuf.at[p],
                                  sems[p][1])
            dr = pltpu.async_copy(rel_hbm.at[ridx.at[c]], rbuf.at[p],
                                  sems[p][2])
            return (dh, dt, dr)

        pend = fire(0, 0)
        for c in range(NCHUNK):
            p = c % 2
            with jax.named_scope(f"dwait{c}"):
                for d in pend:
                    d.wait()
            if c + 1 < NCHUNK:
                pend = fire(c + 1, 1 - p)
            hb, tb, rb = hbuf.at[p], tbuf.at[p], rbuf.at[p]

            with jax.named_scope(f"rows{c}"):
                @plsc.parallel_loop(0, CHUNK, unroll=1)
                def row_fn(i, hb=hb, tb=tb, rb=rb):
                    hv = [hb[i, pl.ds(s * 16, 16)] for s in range(NSUB)]
                    tv = [tb[i, pl.ds(s * 16, 16)] for s in range(NSUB)]
                    rv = [rb[i, pl.ds(s * 16, 16)] for s in range(NSUB)]
                    ih = _rsqrt(_sumsq(hv))
                    it = _rsqrt(_sumsq(tv))
                    ir = _rsqrt(_sumsq(rv))
                    ihv = jnp.full((16,), ih, jnp.float32)
                    itv = jnp.full((16,), it, jnp.float32)
                    irv = jnp.full((16,), ir, jnp.float32)
                    acc = _treesum([
                        jnp.abs(hv[s] * ihv + rv[s] * irv - tv[s] * itv)
                        for s in range(NSUB)
                    ])
                    pb_s[i, pl.ds(0, 16)] = acc

            def grp(b, _, c=c):
                sc = zero
                for j in range(16):
                    jv = jnp.full((16,), j, jnp.int32)
                    sc = sc + plsc.load_gather(pb_s, [b * 16 + lane, jv])
                score[c, pl.ds(b * 16, 16)] = sc
                return 0

            with jax.named_scope(f"tr{c}"):
                lax.fori_loop(0, NBLK, grp, 0)

        pltpu.sync_copy(score, out_hbm.at[wid])

    return trans_score


_sc_score = _make_kernel()


def kernel(batch_h, batch_t, batch_r, tail_emb, rel_emb):
    h = batch_h.astype(jnp.int32).reshape(NW, NCHUNK, CHUNK)
    t = batch_t.astype(jnp.int32).reshape(NW, NCHUNK, CHUNK)
    r = batch_r.astype(jnp.int32).reshape(NW, NCHUNK, CHUNK)
    out = _sc_score(h, t, r, tail_emb, rel_emb)
    return out.reshape(-1)


# speedup vs baseline: 1.2380x; 1.0586x over previous
import functools

import jax
import jax.numpy as jnp
from jax import lax
from jax.experimental import pallas as pl
from jax.experimental.pallas import tpu as pltpu
from jax.experimental.pallas import tpu_sc as plsc

DIM = 128
BATCH = 16384
NW = 32
CHUNK = 128
ROWS_PER_W = BATCH // NW
NCHUNK = ROWS_PER_W // CHUNK
NBLK = CHUNK // 16
NSUB = DIM // 16
NREL = 1000
RELW = 64


def _treesum(xs):
    xs = list(xs)
    while len(xs) > 1:
        nxt = [a + b for a, b in zip(xs[::2], xs[1::2])]
        if len(xs) % 2:
            nxt.append(xs[-1])
        xs = nxt
    return xs[0]


def _rsqrt(x):
    i = lax.bitcast_convert_type(x, jnp.int32)
    i = jnp.int32(0x5F3759DF) - lax.shift_right_arithmetic(i, 1)
    y = lax.bitcast_convert_type(i, jnp.float32)
    xh = jnp.float32(0.5) * x
    for _ in range(3):
        y = y * (jnp.float32(1.5) - xh * y * y)
    return y


def _vrsqrt(x):
    i = lax.bitcast_convert_type(x, jnp.int32)
    i = jnp.full((16,), 0x5F3759DF, jnp.int32) - lax.shift_right_arithmetic(
        i, jnp.full((16,), 1, jnp.int32))
    y = lax.bitcast_convert_type(i, jnp.float32)
    xh = jnp.float32(0.5) * x
    for _ in range(3):
        y = y * (jnp.float32(1.5) - xh * y * y)
    return y


def _sumsq(vs):
    acc = _treesum([v * v for v in vs])
    return jnp.maximum(jnp.sum(acc), jnp.float32(1e-24))


def _make_kernel():
    mesh = plsc.VectorSubcoreMesh(core_axis_name="c", subcore_axis_name="s")

    @functools.partial(
        pl.kernel,
        mesh=mesh,
        compiler_params=pltpu.CompilerParams(needs_layout_passes=False),
        out_type=jax.ShapeDtypeStruct((NW, NCHUNK, CHUNK), jnp.float32),
        scratch_types=[
            pltpu.VMEM((NCHUNK, CHUNK), jnp.int32),
            pltpu.VMEM((NCHUNK, CHUNK), jnp.int32),
            pltpu.VMEM((NCHUNK, CHUNK), jnp.int32),
            pltpu.VMEM((2, CHUNK, DIM), jnp.float32),
            pltpu.VMEM((2, CHUNK, DIM), jnp.float32),
            pltpu.VMEM((2, CHUNK, DIM), jnp.float32),
            pltpu.VMEM((CHUNK, 17), jnp.float32),
            pltpu.VMEM((NCHUNK, CHUNK), jnp.float32),
            pltpu.VMEM((1024,), jnp.float32),
            pltpu.VMEM_SHARED((1024,), jnp.float32),
            pltpu.SemaphoreType.DMA,
            pltpu.SemaphoreType.DMA,
            pltpu.SemaphoreType.DMA,
            pltpu.SemaphoreType.DMA,
            pltpu.SemaphoreType.DMA,
            pltpu.SemaphoreType.DMA,
            pltpu.SemaphoreType.DMA,
        ],
    )
    def trans_score(h_hbm, t_hbm, r_hbm, tail_hbm, rel_hbm, out_hbm,
                    hidx, tidx, ridx, hbuf, tbuf, rbuf, pb_s, score,
                    invr, spm_invr,
                    sh0, st0, sr0, sh1, st1, sr1, sem_p):
        wid = lax.axis_index("s") * 2 + lax.axis_index("c")
        pltpu.sync_copy(h_hbm.at[wid], hidx)
        pltpu.sync_copy(t_hbm.at[wid], tidx)
        pltpu.sync_copy(r_hbm.at[wid], ridx)

        lane = lax.iota(jnp.int32, 16)
        zero = jnp.zeros((16,), jnp.float32)
        sems = ((sh0, st0, sr0), (sh1, st1, sr1))

        def fire(c, p):
            dh = pltpu.async_copy(tail_hbm.at[hidx.at[c]], hbuf.at[p],
                                  sems[p][0])
            dt = pltpu.async_copy(tail_hbm.at[tidx.at[c]], tbuf.at[p],
                                  sems[p][1])
            dr = pltpu.async_copy(rel_hbm.at[ridx.at[c]], rbuf.at[p],
                                  sems[p][2])
            return (dh, dt, dr)

        pend = fire(0, 0)

        tid = lax.axis_index("s")
        rstart = jnp.minimum(tid * RELW, NREL - RELW)
        pltpu.async_copy(rel_hbm.at[pl.ds(rstart, RELW)],
                         hbuf.at[1, pl.ds(0, RELW)], sem_p).wait()
        relb = hbuf.at[1]
        for g in range(RELW // 16):
            def prow(k, _, g=g):
                row = g * 16 + k
                vs = [relb[row, pl.ds(s * 16, 16)] for s in range(NSUB)]
                pb_s[k, pl.ds(0, 16)] = _treesum([v * v for v in vs])
                return 0

            lax.fori_loop(0, 16, prow, 0, unroll=4)
            sm = zero
            for j in range(16):
                jv = jnp.full((16,), j, jnp.int32)
                sm = sm + plsc.load_gather(pb_s, [lane, jv])
            inv = _vrsqrt(jnp.maximum(sm, jnp.float32(1e-24)))
            invr[pl.ds(rstart + g * 16, 16)] = inv
        pltpu.sync_copy(invr.at[pl.ds(rstart, RELW)],
                        spm_invr.at[pl.ds(rstart, RELW)])
        plsc.subcore_barrier()
        pltpu.sync_copy(spm_invr, invr)

        for c in range(NCHUNK):
            p = c % 2
            for d in pend:
                d.wait()
            if c + 1 < NCHUNK:
                pend = fire(c + 1, 1 - p)
            hb, tb, rb = hbuf.at[p], tbuf.at[p], rbuf.at[p]

            @plsc.parallel_loop(0, CHUNK, unroll=1)
            def row_fn(i, hb=hb, tb=tb, rb=rb, c=c):
                hv = [hb[i, pl.ds(s * 16, 16)] for s in range(NSUB)]
                tv = [tb[i, pl.ds(s * 16, 16)] for s in range(NSUB)]
                rv = [rb[i, pl.ds(s * 16, 16)] for s in range(NSUB)]
                ih = _rsqrt(_sumsq(hv))
                it = _rsqrt(_sumsq(tv))
                ihv = jnp.full((16,), ih, jnp.float32)
                itv = jnp.full((16,), it, jnp.float32)
                rix = plsc.load_gather(
                    ridx, [jnp.full((16,), c, jnp.int32),
                           jnp.zeros((16,), jnp.int32) + i])
                irv = plsc.load_gather(invr, [rix])
                acc = _treesum([
                    jnp.abs(hv[s] * ihv + rv[s] * irv - tv[s] * itv)
                    for s in range(NSUB)
                ])
                pb_s[i, pl.ds(0, 16)] = acc

            def grp(b, _, c=c):
                sc = zero
                for j in range(16):
                    jv = jnp.full((16,), j, jnp.int32)
                    sc = sc + plsc.load_gather(pb_s, [b * 16 + lane, jv])
                score[c, pl.ds(b * 16, 16)] = sc
                return 0

            lax.fori_loop(0, NBLK, grp, 0)

        pltpu.sync_copy(score, out_hbm.at[wid])

    return trans_score


_sc_score = _make_kernel()


def kernel(batch_h, batch_t, batch_r, tail_emb, rel_emb):
    h = batch_h.astype(jnp.int32).reshape(NW, NCHUNK, CHUNK)
    t = batch_t.astype(jnp.int32).reshape(NW, NCHUNK, CHUNK)
    r = batch_r.astype(jnp.int32).reshape(NW, NCHUNK, CHUNK)
    out = _sc_score(h, t, r, tail_emb, rel_emb)
    return out.reshape(-1)


# speedup vs baseline: 1.3331x; 1.0768x over previous
import functools

import jax
import jax.numpy as jnp
from jax import lax
from jax.experimental import pallas as pl
from jax.experimental.pallas import tpu as pltpu
from jax.experimental.pallas import tpu_sc as plsc

DIM = 128
BATCH = 16384
NW = 32
CHUNK = 128
ROWS_PER_W = BATCH // NW
NCHUNK = ROWS_PER_W // CHUNK
NBLK = CHUNK // 16
NSUB = DIM // 16
NREL = 1000
RELW = 64


def _treesum(xs):
    xs = list(xs)
    while len(xs) > 1:
        nxt = [a + b for a, b in zip(xs[::2], xs[1::2])]
        if len(xs) % 2:
            nxt.append(xs[-1])
        xs = nxt
    return xs[0]


def _rsqrt(x):
    i = lax.bitcast_convert_type(x, jnp.int32)
    i = jnp.int32(0x5F3759DF) - lax.shift_right_arithmetic(i, 1)
    y = lax.bitcast_convert_type(i, jnp.float32)
    xh = jnp.float32(0.5) * x
    for _ in range(2):
        y = y * (jnp.float32(1.5) - xh * y * y)
    return y


def _vrsqrt(x):
    i = lax.bitcast_convert_type(x, jnp.int32)
    i = jnp.full((16,), 0x5F3759DF, jnp.int32) - lax.shift_right_arithmetic(
        i, jnp.full((16,), 1, jnp.int32))
    y = lax.bitcast_convert_type(i, jnp.float32)
    xh = jnp.float32(0.5) * x
    for _ in range(3):
        y = y * (jnp.float32(1.5) - xh * y * y)
    return y


def _sumsq(vs):
    acc = _treesum([v * v for v in vs])
    return jnp.maximum(jnp.sum(acc), jnp.float32(1e-24))


def _make_kernel():
    mesh = plsc.VectorSubcoreMesh(core_axis_name="c", subcore_axis_name="s")

    @functools.partial(
        pl.kernel,
        mesh=mesh,
        compiler_params=pltpu.CompilerParams(needs_layout_passes=False),
        out_type=jax.ShapeDtypeStruct((BATCH,), jnp.float32),
        scratch_types=[
            pltpu.VMEM((ROWS_PER_W,), jnp.int32),
            pltpu.VMEM((ROWS_PER_W,), jnp.int32),
            pltpu.VMEM((ROWS_PER_W,), jnp.int32),
            pltpu.VMEM((2, CHUNK, DIM), jnp.float32),
            pltpu.VMEM((2, CHUNK, DIM), jnp.float32),
            pltpu.VMEM((2, CHUNK, DIM), jnp.float32),
            pltpu.VMEM((CHUNK, 17), jnp.float32),
            pltpu.VMEM((ROWS_PER_W,), jnp.float32),
            pltpu.VMEM((1024,), jnp.float32),
            pltpu.VMEM_SHARED((1024,), jnp.float32),
            pltpu.SemaphoreType.DMA,
            pltpu.SemaphoreType.DMA,
            pltpu.SemaphoreType.DMA,
            pltpu.SemaphoreType.DMA,
            pltpu.SemaphoreType.DMA,
            pltpu.SemaphoreType.DMA,
            pltpu.SemaphoreType.DMA,
        ],
    )
    def trans_score(h_hbm, t_hbm, r_hbm, tail_hbm, rel_hbm, out_hbm,
                    hidx, tidx, ridx, hbuf, tbuf, rbuf, pb_s, score,
                    invr, spm_invr,
                    sh0, st0, sr0, sh1, st1, sr1, sem_p):
        wid = lax.axis_index("s") * 2 + lax.axis_index("c")
        base = wid * ROWS_PER_W
        di1 = pltpu.async_copy(h_hbm.at[pl.ds(base, ROWS_PER_W)], hidx, sh1)
        di2 = pltpu.async_copy(t_hbm.at[pl.ds(base, ROWS_PER_W)], tidx, st1)
        di3 = pltpu.async_copy(r_hbm.at[pl.ds(base, ROWS_PER_W)], ridx, sr1)
        di1.wait()
        di2.wait()
        di3.wait()

        lane = lax.iota(jnp.int32, 16)
        zero = jnp.zeros((16,), jnp.float32)
        sems = ((sh0, st0, sr0), (sh1, st1, sr1))

        def fire(c, p):
            sl = pl.ds(c * CHUNK, CHUNK)
            dh = pltpu.async_copy(tail_hbm.at[hidx.at[sl]], hbuf.at[p],
                                  sems[p][0])
            dt = pltpu.async_copy(tail_hbm.at[tidx.at[sl]], tbuf.at[p],
                                  sems[p][1])
            dr = pltpu.async_copy(rel_hbm.at[ridx.at[sl]], rbuf.at[p],
                                  sems[p][2])
            return (dh, dt, dr)

        pend = fire(0, 0)

        tid = lax.axis_index("s")
        rstart = jnp.minimum(tid * RELW, NREL - RELW)
        pltpu.async_copy(rel_hbm.at[pl.ds(rstart, RELW)],
                         hbuf.at[1, pl.ds(0, RELW)], sem_p).wait()
        relb = hbuf.at[1]
        for g in range(RELW // 16):
            def prow(k, _, g=g):
                row = g * 16 + k
                vs = [relb[row, pl.ds(s * 16, 16)] for s in range(NSUB)]
                pb_s[k, pl.ds(0, 16)] = _treesum([v * v for v in vs])
                return 0

            lax.fori_loop(0, 16, prow, 0, unroll=4)
            sm = zero
            for j in range(16):
                jv = jnp.full((16,), j, jnp.int32)
                sm = sm + plsc.load_gather(pb_s, [lane, jv])
            inv = _vrsqrt(jnp.maximum(sm, jnp.float32(1e-24)))
            invr[pl.ds(rstart + g * 16, 16)] = inv
        pltpu.sync_copy(invr.at[pl.ds(rstart, RELW)],
                        spm_invr.at[pl.ds(rstart, RELW)])
        plsc.subcore_barrier()
        pltpu.sync_copy(spm_invr, invr)

        for c in range(NCHUNK):
            p = c % 2
            if c + 1 < NCHUNK:
                pend_next = fire(c + 1, 1 - p)
            for d in pend:
                d.wait()
            if c + 1 < NCHUNK:
                pend = pend_next
            hb, tb, rb = hbuf.at[p], tbuf.at[p], rbuf.at[p]

            @plsc.parallel_loop(0, CHUNK, unroll=1)
            def row_fn(i, hb=hb, tb=tb, rb=rb, c=c):
                hv = [hb[i, pl.ds(s * 16, 16)] for s in range(NSUB)]
                tv = [tb[i, pl.ds(s * 16, 16)] for s in range(NSUB)]
                rv = [rb[i, pl.ds(s * 16, 16)] for s in range(NSUB)]
                sh = _sumsq(hv)
                ih = _rsqrt(sh)
                it = _rsqrt(_sumsq(tv))
                q = sh * ih
                ihv = jnp.full((16,), ih, jnp.float32)
                bv = jnp.full((16,), q * it, jnp.float32)
                rix = plsc.load_gather(
                    ridx, [jnp.zeros((16,), jnp.int32) + (c * CHUNK + i)])
                av = plsc.load_gather(invr, [rix]) * jnp.full((16,), q,
                                                              jnp.float32)
                acc = _treesum([
                    jnp.abs(hv[s] + rv[s] * av - tv[s] * bv)
                    for s in range(NSUB)
                ])
                pb_s[i, pl.ds(0, 16)] = acc * ihv

            def grp(b, _, c=c):
                sc = zero
                for j in range(16):
                    jv = jnp.full((16,), j, jnp.int32)
                    sc = sc + plsc.load_gather(pb_s, [b * 16 + lane, jv])
                score[pl.ds(c * CHUNK + b * 16, 16)] = sc
                return 0

            lax.fori_loop(0, NBLK, grp, 0, unroll=2)

        pltpu.sync_copy(score, out_hbm.at[pl.ds(base, ROWS_PER_W)])

    return trans_score


_sc_score = _make_kernel()


def kernel(batch_h, batch_t, batch_r, tail_emb, rel_emb):
    return _sc_score(batch_h.astype(jnp.int32), batch_t.astype(jnp.int32),
                     batch_r.astype(jnp.int32), tail_emb, rel_emb)


# speedup vs baseline: 1.3373x; 1.0032x over previous
import functools

import jax
import jax.numpy as jnp
from jax import lax
from jax.experimental import pallas as pl
from jax.experimental.pallas import tpu as pltpu
from jax.experimental.pallas import tpu_sc as plsc

DIM = 128
BATCH = 16384
NW = 32
CHUNK = 128
ROWS_PER_W = BATCH // NW
NCHUNK = ROWS_PER_W // CHUNK
NBLK = CHUNK // 16
NSUB = DIM // 16
NREL = 1000
RELW = 64


def _treesum(xs):
    xs = list(xs)
    while len(xs) > 1:
        nxt = [a + b for a, b in zip(xs[::2], xs[1::2])]
        if len(xs) % 2:
            nxt.append(xs[-1])
        xs = nxt
    return xs[0]


def _rsqrt(x):
    i = lax.bitcast_convert_type(x, jnp.int32)
    i = jnp.int32(0x5F3759DF) - lax.shift_right_arithmetic(i, 1)
    y = lax.bitcast_convert_type(i, jnp.float32)
    xh = jnp.float32(0.5) * x
    for _ in range(2):
        y = y * (jnp.float32(1.5) - xh * y * y)
    return y


def _vrsqrt(x):
    i = lax.bitcast_convert_type(x, jnp.int32)
    i = jnp.full((16,), 0x5F3759DF, jnp.int32) - lax.shift_right_arithmetic(
        i, jnp.full((16,), 1, jnp.int32))
    y = lax.bitcast_convert_type(i, jnp.float32)
    xh = jnp.float32(0.5) * x
    for _ in range(3):
        y = y * (jnp.float32(1.5) - xh * y * y)
    return y


def _sumsq(vs):
    acc = _treesum([v * v for v in vs])
    return jnp.maximum(jnp.sum(acc), jnp.float32(1e-24))


def _make_kernel():
    mesh = plsc.VectorSubcoreMesh(core_axis_name="c", subcore_axis_name="s")

    @functools.partial(
        pl.kernel,
        mesh=mesh,
        compiler_params=pltpu.CompilerParams(needs_layout_passes=False),
        out_type=jax.ShapeDtypeStruct((BATCH,), jnp.float32),
        scratch_types=[
            pltpu.VMEM((ROWS_PER_W,), jnp.int32),
            pltpu.VMEM((ROWS_PER_W,), jnp.int32),
            pltpu.VMEM((ROWS_PER_W,), jnp.int32),
            pltpu.VMEM((2, CHUNK, DIM), jnp.float32),
            pltpu.VMEM((2, CHUNK, DIM), jnp.float32),
            pltpu.VMEM((2, CHUNK, DIM), jnp.float32),
            pltpu.VMEM((CHUNK, 17), jnp.float32),
            pltpu.VMEM((ROWS_PER_W,), jnp.float32),
            pltpu.VMEM((1024,), jnp.float32),
            pltpu.VMEM_SHARED((1024,), jnp.float32),
            pltpu.SemaphoreType.DMA,
            pltpu.SemaphoreType.DMA,
            pltpu.SemaphoreType.DMA,
            pltpu.SemaphoreType.DMA,
            pltpu.SemaphoreType.DMA,
            pltpu.SemaphoreType.DMA,
            pltpu.SemaphoreType.DMA,
        ],
    )
    def trans_score(h_hbm, t_hbm, r_hbm, tail_hbm, rel_hbm, out_hbm,
                    hidx, tidx, ridx, hbuf, tbuf, rbuf, pb_s, score,
                    invr, spm_invr,
                    sh0, st0, sr0, sh1, st1, sr1, sem_p):
        wid = lax.axis_index("s") * 2 + lax.axis_index("c")
        base = wid * ROWS_PER_W
        di1 = pltpu.async_copy(h_hbm.at[pl.ds(base, ROWS_PER_W)], hidx, sh1)
        di2 = pltpu.async_copy(t_hbm.at[pl.ds(base, ROWS_PER_W)], tidx, st1)
        di3 = pltpu.async_copy(r_hbm.at[pl.ds(base, ROWS_PER_W)], ridx, sr1)
        di1.wait()
        di2.wait()
        di3.wait()

        lane = lax.iota(jnp.int32, 16)
        zero = jnp.zeros((16,), jnp.float32)
        sems = ((sh0, st0, sr0), (sh1, st1, sr1))

        def fire(c, p):
            sl = pl.ds(c * CHUNK, CHUNK)
            dh = pltpu.async_copy(tail_hbm.at[hidx.at[sl]], hbuf.at[p],
                                  sems[p][0])
            dt = pltpu.async_copy(tail_hbm.at[tidx.at[sl]], tbuf.at[p],
                                  sems[p][1])
            dr = pltpu.async_copy(rel_hbm.at[ridx.at[sl]], rbuf.at[p],
                                  sems[p][2])
            return (dh, dt, dr)

        pend = fire(0, 0)

        tid = lax.axis_index("s")
        rstart = jnp.minimum(tid * RELW, NREL - RELW)
        pltpu.async_copy(rel_hbm.at[pl.ds(rstart, RELW)],
                         hbuf.at[1, pl.ds(0, RELW)], sem_p).wait()
        relb = hbuf.at[1]
        for g in range(RELW // 16):
            def prow(k, _, g=g):
                row = g * 16 + k
                vs = [relb[row, pl.ds(s * 16, 16)] for s in range(NSUB)]
                pb_s[k, pl.ds(0, 16)] = _treesum([v * v for v in vs])
                return 0

            lax.fori_loop(0, 16, prow, 0, unroll=4)
            sm = zero
            for j in range(16):
                jv = jnp.full((16,), j, jnp.int32)
                sm = sm + plsc.load_gather(pb_s, [lane, jv])
            inv = _vrsqrt(jnp.maximum(sm, jnp.float32(1e-24)))
            invr[pl.ds(rstart + g * 16, 16)] = inv
        pltpu.sync_copy(invr.at[pl.ds(rstart, RELW)],
                        spm_invr.at[pl.ds(rstart, RELW)])
        plsc.subcore_barrier()
        pltpu.sync_copy(spm_invr, invr)

        for c in range(NCHUNK):
            p = c % 2
            if c + 1 < NCHUNK:
                pend_next = fire(c + 1, 1 - p)
            for d in pend:
                d.wait()
            if c + 1 < NCHUNK:
                pend = pend_next
            hb, tb, rb = hbuf.at[p], tbuf.at[p], rbuf.at[p]

            @plsc.parallel_loop(0, CHUNK, unroll=1)
            def row_fn(i, hb=hb, tb=tb, rb=rb, c=c):
                hv = [hb[i, pl.ds(s * 16, 16)] for s in range(NSUB)]
                tv = [tb[i, pl.ds(s * 16, 16)] for s in range(NSUB)]
                rv = [rb[i, pl.ds(s * 16, 16)] for s in range(NSUB)]
                sh = _sumsq(hv)
                ih = _rsqrt(sh)
                it = _rsqrt(_sumsq(tv))
                q = sh * ih
                ihv = jnp.full((16,), ih, jnp.float32)
                bv = jnp.full((16,), q * it, jnp.float32)
                rix = plsc.load_gather(
                    ridx, [jnp.zeros((16,), jnp.int32) + (c * CHUNK + i)])
                av = plsc.load_gather(invr, [rix]) * jnp.full((16,), q,
                                                              jnp.float32)
                acc = _treesum([
                    jnp.abs(hv[s] + rv[s] * av - tv[s] * bv)
                    for s in range(NSUB)
                ])
                pb_s[i, pl.ds(0, 16)] = acc * ihv

            def grp(b, _, c=c):
                rows = b * 16 + lane
                sc = _treesum([
                    plsc.load_gather(pb_s,
                                     [rows, jnp.full((16,), j, jnp.int32)])
                    for j in range(16)
                ])
                score[pl.ds(c * CHUNK + b * 16, 16)] = sc
                return 0

            lax.fori_loop(0, NBLK, grp, 0, unroll=2)

        pltpu.sync_copy(score, out_hbm.at[pl.ds(base, ROWS_PER_W)])

    return trans_score


_sc_score = _make_kernel()


def kernel(batch_h, batch_t, batch_r, tail_emb, rel_emb):
    return _sc_score(batch_h.astype(jnp.int32), batch_t.astype(jnp.int32),
                     batch_r.astype(jnp.int32), tail_emb, rel_emb)
